# baseline (speedup 1.0000x reference)
import jax
import jax.numpy as jnp
from jax import lax
from jax.experimental import pallas as pl
from jax.experimental.pallas import tpu as pltpu
from jax.experimental.pallas import tpu_sc as plsc

B = 16384
EMB = 32
N_GENRES = 4
TEXT_LEN = 20

NUM_WORKERS = 32
ROWS_PER_WORKER = B // NUM_WORKERS
CHUNK = 64
NGROUPS = CHUNK // 16
NCHUNKS = ROWS_PER_WORKER // CHUNK
TOK_PER_CHUNK = CHUNK * TEXT_LEN
IDX_W = 128
NGATHER = TOK_PER_CHUNK // IDX_W
NGG = CHUNK * N_GENRES // IDX_W
IVW = 17


def _tree_sum(vals):
    while len(vals) > 1:
        nxt = [vals[i] + vals[i + 1] for i in range(0, len(vals) - 1, 2)]
        if len(vals) % 2:
            nxt.append(vals[-1])
        vals = nxt
    return vals[0]


def _body(title_idx, genres_bf, text_idxf,
          title_tab, genre_tab, text_tab, out,
          tidx_v, gidx_v, xidx_v, trows_v, grows_v, xrows_v,
          invb_v, zb_v, r0_v, out_v, sem_s, sem_g, sem_o):
    wid = lax.axis_index("s") * 2 + lax.axis_index("c")
    base = wid * ROWS_PER_WORKER

    lane = jax.lax.iota(jnp.int32, 16)

    pltpu.sync_copy(text_tab.at[pl.ds(0, 8)], r0_v)
    r0 = [r0_v[0, pl.ds(h, 16)] for h in (0, 16)]

    def fire_stage(c):
        p = c % 2
        rb = base + c * CHUNK
        cps = [
            pltpu.async_copy(title_idx.at[pl.ds(rb, CHUNK)],
                             tidx_v[p], sem_s[p]),
        ]
        for j in range(NGG):
            cps.append(pltpu.async_copy(
                genres_bf.at[pl.ds(rb * N_GENRES + j * IDX_W, IDX_W)],
                gidx_v[p].at[j], sem_s[p]))
        for j in range(NGATHER):
            cps.append(pltpu.async_copy(
                text_idxf.at[pl.ds(rb * TEXT_LEN + j * IDX_W, IDX_W)],
                xidx_v[p].at[j], sem_s[p]))
        return cps

    def fire_gathers(c):
        p = c % 2
        gcps = [pltpu.async_copy(title_tab.at[tidx_v[p]],
                                 trows_v[p], sem_g[p])]
        for j in range(NGG):
            gcps.append(pltpu.async_copy(
                genre_tab.at[gidx_v[p].at[j]],
                grows_v[p].at[pl.ds(j * IDX_W, IDX_W)], sem_g[p]))
        for j in range(NGATHER):
            gcps.append(pltpu.async_copy(
                text_tab.at[xidx_v[p].at[j]],
                xrows_v[p].at[pl.ds(j * IDX_W, IDX_W)], sem_g[p]))
        return gcps

    def compute(c):
        p = c % 2

        def group_body(g, group_carry):
            trow16 = (lane + g * 16) * TEXT_LEN
            ws = []
            for t in range(TEXT_LEN):
                pos = trow16 + t
                tok = plsc.load_gather(xidx_v[p], [pos >> 7, pos & 127])
                ws.append(jnp.where(tok != 0, 1.0, 0.0).astype(jnp.float32))
            cnt = _tree_sum(ws)
            inv = jnp.float32(1.0) / jnp.maximum(cnt, jnp.float32(1.0))
            nz = jnp.float32(TEXT_LEN) - cnt
            ob = (lane + g * 16) * IVW
            for k in range(16):
                plsc.store_scatter(invb_v[p], [ob + k], inv)
                plsc.store_scatter(zb_v[p], [ob + k], nz)
            return group_carry

        lax.fori_loop(0, NGROUPS, group_body, None)

        def row_body(b, row_carry):
            iv = invb_v[p][pl.ds(b * IVW, 16)]
            nz = zb_v[p][pl.ds(b * IVW, 16)]
            for hi, h in enumerate((0, 16)):
                tv = trows_v[p][b, pl.ds(h, 16)]
                out_v_p = out_v[p]
                out_v_p[pl.ds(b * 3 * EMB + h, 16)] = tv
                ga = _tree_sum([grows_v[p][b * N_GENRES + j, pl.ds(h, 16)]
                                for j in range(N_GENRES)])
                out_v_p[pl.ds(b * 3 * EMB + EMB + h, 16)] = ga * 0.25
                xa = _tree_sum([xrows_v[p][b * TEXT_LEN + t, pl.ds(h, 16)]
                                for t in range(TEXT_LEN)])
                out_v_p[pl.ds(b * 3 * EMB + 2 * EMB + h, 16)] = (
                    (xa - nz * r0[hi]) * iv)
            return row_carry

        lax.fori_loop(0, CHUNK, row_body, None, unroll=2)

    def fire_out(c):
        p = c % 2
        rb = base + c * CHUNK
        return [pltpu.async_copy(
            out_v[p], out.at[pl.ds(rb * 3 * EMB, CHUNK * 3 * EMB)],
            sem_o[p])]

    stage_cps = {0: fire_stage(0)}
    for cp in stage_cps[0]:
        cp.wait()
    gather_cps = {0: fire_gathers(0)}
    stage_cps[1] = fire_stage(1)
    out_cps = {}
    for c in range(NCHUNKS):
        if c + 1 < NCHUNKS:
            for cp in stage_cps[c + 1]:
                cp.wait()
            gather_cps[c + 1] = fire_gathers(c + 1)
        for cp in gather_cps[c]:
            cp.wait()
        if c >= 2:
            for cp in out_cps[c - 2]:
                cp.wait()
        compute(c)
        out_cps[c] = fire_out(c)
        if c + 2 < NCHUNKS:
            stage_cps[c + 2] = fire_stage(c + 2)
    for cp in out_cps[NCHUNKS - 2] + out_cps[NCHUNKS - 1]:
        cp.wait()


@jax.jit
def _run(title_idx, genres_bf, text_idxf, title_tab, genre_tab, text_tab):
    mesh = plsc.VectorSubcoreMesh(core_axis_name="c", subcore_axis_name="s")
    fn = pl.kernel(
        _body,
        out_type=jax.ShapeDtypeStruct((B * 3 * EMB,), jnp.float32),
        mesh=mesh,
        scratch_types=[
            [pltpu.VMEM((CHUNK,), jnp.int32)] * 2,
            [pltpu.VMEM((NGG, IDX_W), jnp.int32)] * 2,
            [pltpu.VMEM((NGATHER, IDX_W), jnp.int32)] * 2,
            [pltpu.VMEM((CHUNK, EMB), jnp.float32)] * 2,
            [pltpu.VMEM((CHUNK * N_GENRES, EMB), jnp.float32)] * 2,
            [pltpu.VMEM((TOK_PER_CHUNK, EMB), jnp.float32)] * 2,
            [pltpu.VMEM((CHUNK * IVW,), jnp.float32)] * 2,
            [pltpu.VMEM((CHUNK * IVW,), jnp.float32)] * 2,
            pltpu.VMEM((8, EMB), jnp.float32),
            [pltpu.VMEM((CHUNK * 3 * EMB,), jnp.float32)] * 2,
            [pltpu.SemaphoreType.DMA] * 2,
            [pltpu.SemaphoreType.DMA] * 2,
            [pltpu.SemaphoreType.DMA] * 2,
        ],
        compiler_params=pltpu.CompilerParams(needs_layout_passes=False,
                                             use_tc_tiling_on_sc=False),
    )
    return fn(title_idx, genres_bf, text_idxf, title_tab, genre_tab, text_tab)


def kernel(movie_title, movie_genres, movie_title_text,
           title_table, genre_table, text_table):
    title_idx = movie_title.astype(jnp.int32)
    m = jnp.int32(0x7FFFFFFF)
    genres_bf = (movie_genres.astype(jnp.int32) & m).reshape(-1)
    text_idxf = (movie_title_text.astype(jnp.int32) & m).reshape(-1)
    flat = _run(title_idx, genres_bf, text_idxf,
                title_table, genre_table, text_table)
    return jnp.maximum(flat.reshape(B, 3 * EMB), jnp.float32(-3.0e38))


# speedup vs baseline: 1.1160x; 1.1160x over previous
import jax
import jax.numpy as jnp
from jax import lax
from jax.experimental import pallas as pl
from jax.experimental.pallas import tpu as pltpu
from jax.experimental.pallas import tpu_sc as plsc

B = 16384
EMB = 32
N_GENRES = 4
TEXT_LEN = 20

NUM_WORKERS = 32
ROWS_PER_WORKER = B // NUM_WORKERS
CHUNK = 64
NGROUPS = CHUNK // 16
NCHUNKS = ROWS_PER_WORKER // CHUNK
TOK_PER_CHUNK = CHUNK * TEXT_LEN
IDX_W = 128
NGATHER = TOK_PER_CHUNK // IDX_W
NGG = CHUNK * N_GENRES // IDX_W
IVW = 17


def _tree_sum(vals):
    while len(vals) > 1:
        nxt = [vals[i] + vals[i + 1] for i in range(0, len(vals) - 1, 2)]
        if len(vals) % 2:
            nxt.append(vals[-1])
        vals = nxt
    return vals[0]


def _body(title_idx, genres_bf, text_idxf,
          title_tab, genre_tab, text_tab, out,
          tidx_v, gidx_v, xidx_v, trows_v, grows_v, xrows_v,
          invb_v, zb_v, r0_v, stab_v, out_v, sem_s, sem_g, sem_t, sem_o):
    wid = lax.axis_index("s") * 2 + lax.axis_index("c")
    base = wid * ROWS_PER_WORKER

    lane = jax.lax.iota(jnp.int32, 16)

    pltpu.sync_copy(text_tab.at[pl.ds(0, 8)], r0_v)
    r0 = [r0_v[0, pl.ds(h, 16)] for h in (0, 16)]

    @pl.when(lax.axis_index("s") == 0)
    def _stage_shared():
        pltpu.sync_copy(text_tab, stab_v)

    plsc.subcore_barrier()

    def fire_stage(c):
        p = c % 2
        rb = base + c * CHUNK
        cps = [
            pltpu.async_copy(title_idx.at[pl.ds(rb, CHUNK)],
                             tidx_v[p], sem_s[p]),
        ]
        for j in range(NGG):
            cps.append(pltpu.async_copy(
                genres_bf.at[pl.ds(rb * N_GENRES + j * IDX_W, IDX_W)],
                gidx_v[p].at[j], sem_s[p]))
        for j in range(NGATHER):
            cps.append(pltpu.async_copy(
                text_idxf.at[pl.ds(rb * TEXT_LEN + j * IDX_W, IDX_W)],
                xidx_v[p].at[j], sem_s[p]))
        return cps

    def fire_text_gathers(c):
        p = c % 2
        gcps = []
        for j in range(NGATHER):
            gcps.append(pltpu.async_copy(
                stab_v.at[xidx_v[p].at[j]],
                xrows_v[p].at[pl.ds(j * IDX_W, IDX_W)], sem_g[p]))
        return gcps

    def fire_tg_gathers(c):
        p = c % 2
        gcps = [pltpu.async_copy(title_tab.at[tidx_v[p]], trows_v, sem_t)]
        for j in range(NGG):
            gcps.append(pltpu.async_copy(
                genre_tab.at[gidx_v[p].at[j]],
                grows_v.at[pl.ds(j * IDX_W, IDX_W)], sem_t))
        return gcps

    def compute(c):
        p = c % 2

        def group_body(g, group_carry):
            trow16 = (lane + g * 16) * TEXT_LEN
            ws = []
            for t in range(TEXT_LEN):
                pos = trow16 + t
                tok = plsc.load_gather(xidx_v[p], [pos >> 7, pos & 127])
                ws.append(jnp.where(tok != 0, 1.0, 0.0).astype(jnp.float32))
            cnt = _tree_sum(ws)
            inv = jnp.float32(1.0) / jnp.maximum(cnt, jnp.float32(1.0))
            nz = jnp.float32(TEXT_LEN) - cnt
            ob = (lane + g * 16) * IVW
            for k in range(16):
                plsc.store_scatter(invb_v, [ob + k], inv)
                plsc.store_scatter(zb_v, [ob + k], nz)
            return group_carry

        lax.fori_loop(0, NGROUPS, group_body, None)

        def row_body(b, row_carry):
            iv = invb_v[pl.ds(b * IVW, 16)]
            nz = zb_v[pl.ds(b * IVW, 16)]
            for hi, h in enumerate((0, 16)):
                tv = trows_v[b, pl.ds(h, 16)]
                out_v_p = out_v[p]
                out_v_p[pl.ds(b * 3 * EMB + h, 16)] = tv
                ga = _tree_sum([grows_v[b * N_GENRES + j, pl.ds(h, 16)]
                                for j in range(N_GENRES)])
                out_v_p[pl.ds(b * 3 * EMB + EMB + h, 16)] = ga * 0.25
                xa = _tree_sum([xrows_v[p][b * TEXT_LEN + t, pl.ds(h, 16)]
                                for t in range(TEXT_LEN)])
                out_v_p[pl.ds(b * 3 * EMB + 2 * EMB + h, 16)] = (
                    (xa - nz * r0[hi]) * iv)
            return row_carry

        lax.fori_loop(0, CHUNK, row_body, None, unroll=2)

    def fire_out(c):
        p = c % 2
        rb = base + c * CHUNK
        return [pltpu.async_copy(
            out_v[p], out.at[pl.ds(rb * 3 * EMB, CHUNK * 3 * EMB)],
            sem_o[p])]

    stage_cps = {0: fire_stage(0)}
    for cp in stage_cps[0]:
        cp.wait()
    gather_cps = {0: fire_text_gathers(0)}
    tg_cps = {0: fire_tg_gathers(0)}
    stage_cps[1] = fire_stage(1)
    out_cps = {}
    for c in range(NCHUNKS):
        if c + 1 < NCHUNKS:
            for cp in stage_cps[c + 1]:
                cp.wait()
            gather_cps[c + 1] = fire_text_gathers(c + 1)
        for cp in gather_cps[c] + tg_cps[c]:
            cp.wait()
        if c >= 2:
            for cp in out_cps[c - 2]:
                cp.wait()
        compute(c)
        out_cps[c] = fire_out(c)
        if c + 2 < NCHUNKS:
            stage_cps[c + 2] = fire_stage(c + 2)
        if c + 1 < NCHUNKS:
            tg_cps[c + 1] = fire_tg_gathers(c + 1)
    for cp in out_cps[NCHUNKS - 2] + out_cps[NCHUNKS - 1]:
        cp.wait()


@jax.jit
def _run(title_idx, genres_bf, text_idxf, title_tab, genre_tab, text_tab):
    mesh = plsc.VectorSubcoreMesh(core_axis_name="c", subcore_axis_name="s")
    fn = pl.kernel(
        _body,
        out_type=jax.ShapeDtypeStruct((B * 3 * EMB,), jnp.float32),
        mesh=mesh,
        scratch_types=[
            [pltpu.VMEM((CHUNK,), jnp.int32)] * 2,
            [pltpu.VMEM((NGG, IDX_W), jnp.int32)] * 2,
            [pltpu.VMEM((NGATHER, IDX_W), jnp.int32)] * 2,
            pltpu.VMEM((CHUNK, EMB), jnp.float32),
            pltpu.VMEM((CHUNK * N_GENRES, EMB), jnp.float32),
            [pltpu.VMEM((TOK_PER_CHUNK, EMB), jnp.float32)] * 2,
            pltpu.VMEM((CHUNK * IVW,), jnp.float32),
            pltpu.VMEM((CHUNK * IVW,), jnp.float32),
            pltpu.VMEM((8, EMB), jnp.float32),
            pltpu.VMEM_SHARED((10000, EMB), jnp.float32),
            [pltpu.VMEM((CHUNK * 3 * EMB,), jnp.float32)] * 2,
            [pltpu.SemaphoreType.DMA] * 2,
            [pltpu.SemaphoreType.DMA] * 2,
            pltpu.SemaphoreType.DMA,
            [pltpu.SemaphoreType.DMA] * 2,
        ],
        compiler_params=pltpu.CompilerParams(needs_layout_passes=False,
                                             use_tc_tiling_on_sc=False),
    )
    return fn(title_idx, genres_bf, text_idxf, title_tab, genre_tab, text_tab)


def kernel(movie_title, movie_genres, movie_title_text,
           title_table, genre_table, text_table):
    title_idx = movie_title.astype(jnp.int32)
    genres_bf = movie_genres.astype(jnp.int32).reshape(-1)
    text_idxf = movie_title_text.astype(jnp.int32).reshape(-1)
    flat = _run(title_idx, genres_bf, text_idxf,
                title_table, genre_table, text_table)
    return flat.reshape(B, 3 * EMB)


# speedup vs baseline: 1.1325x; 1.0148x over previous
import jax
import jax.numpy as jnp
from jax import lax
from jax.experimental import pallas as pl
from jax.experimental.pallas import tpu as pltpu
from jax.experimental.pallas import tpu_sc as plsc

B = 16384
EMB = 32
N_GENRES = 4
TEXT_LEN = 20

NUM_WORKERS = 32
ROWS_PER_WORKER = B // NUM_WORKERS
CHUNK = 64
NGROUPS = CHUNK // 16
NCHUNKS = ROWS_PER_WORKER // CHUNK
TOK_PER_CHUNK = CHUNK * TEXT_LEN
IDX_W = 128
NGATHER = TOK_PER_CHUNK // IDX_W
NGG = CHUNK * N_GENRES // IDX_W
IVW = 17


def _tree_sum(vals):
    while len(vals) > 1:
        nxt = [vals[i] + vals[i + 1] for i in range(0, len(vals) - 1, 2)]
        if len(vals) % 2:
            nxt.append(vals[-1])
        vals = nxt
    return vals[0]


def _body(title_idx, genres_bf, text_idxf,
          title_tab, genre_tab, text_tab, out,
          tidx_v, gidx_v, xidx_v, trows_v, grows_v, xrows_v,
          invb_v, zb_v, r0_v, stab_v, out_v, sem_s, sem_g, sem_t, sem_o):
    wid = lax.axis_index("s") * 2 + lax.axis_index("c")
    base = wid * ROWS_PER_WORKER

    lane = jax.lax.iota(jnp.int32, 16)

    pltpu.sync_copy(text_tab.at[pl.ds(0, 8)], r0_v)
    r0 = [r0_v[0, pl.ds(h, 16)] for h in (0, 16)]

    @pl.when(lax.axis_index("s") == 0)
    def _stage_shared():
        pltpu.sync_copy(text_tab, stab_v)

    plsc.subcore_barrier()

    def fire_stage(c, p):
        rb = base + c * CHUNK
        cps = [
            pltpu.async_copy(title_idx.at[pl.ds(rb, CHUNK)],
                             tidx_v[p], sem_s[p]),
        ]
        for j in range(NGG):
            cps.append(pltpu.async_copy(
                genres_bf.at[pl.ds(rb * N_GENRES + j * IDX_W, IDX_W)],
                gidx_v[p].at[j], sem_s[p]))
        for j in range(NGATHER):
            cps.append(pltpu.async_copy(
                text_idxf.at[pl.ds(rb * TEXT_LEN + j * IDX_W, IDX_W)],
                xidx_v[p].at[j], sem_s[p]))
        return cps

    def fire_text_gathers(p):
        gcps = []
        for j in range(NGATHER):
            gcps.append(pltpu.async_copy(
                stab_v.at[xidx_v[p].at[j]],
                xrows_v[p].at[pl.ds(j * IDX_W, IDX_W)], sem_g[p]))
        return gcps

    def fire_tg_gathers(p):
        gcps = [pltpu.async_copy(title_tab.at[tidx_v[p]], trows_v, sem_t)]
        for j in range(NGG):
            gcps.append(pltpu.async_copy(
                genre_tab.at[gidx_v[p].at[j]],
                grows_v.at[pl.ds(j * IDX_W, IDX_W)], sem_t))
        return gcps

    def compute(p):

        def group_body(g, group_carry):
            trow16 = (lane + g * 16) * TEXT_LEN
            ws = []
            for t in range(TEXT_LEN):
                pos = trow16 + t
                tok = plsc.load_gather(xidx_v[p], [pos >> 7, pos & 127])
                ws.append(jnp.where(tok != 0, 1.0, 0.0).astype(jnp.float32))
            cnt = _tree_sum(ws)
            inv = jnp.float32(1.0) / jnp.maximum(cnt, jnp.float32(1.0))
            nz = jnp.float32(TEXT_LEN) - cnt
            ob = (lane + g * 16) * IVW
            for k in range(16):
                plsc.store_scatter(invb_v, [ob + k], inv)
                plsc.store_scatter(zb_v, [ob + k], nz)
            return group_carry

        lax.fori_loop(0, NGROUPS, group_body, None)

        def row_body(b, row_carry):
            iv = invb_v[pl.ds(b * IVW, 16)]
            nz = zb_v[pl.ds(b * IVW, 16)]
            for hi, h in enumerate((0, 16)):
                tv = trows_v[b, pl.ds(h, 16)]
                out_v_p = out_v[p]
                out_v_p[pl.ds(b * 3 * EMB + h, 16)] = tv
                ga = _tree_sum([grows_v[b * N_GENRES + j, pl.ds(h, 16)]
                                for j in range(N_GENRES)])
                out_v_p[pl.ds(b * 3 * EMB + EMB + h, 16)] = ga * 0.25
                xa = _tree_sum([xrows_v[p][b * TEXT_LEN + t, pl.ds(h, 16)]
                                for t in range(TEXT_LEN)])
                out_v_p[pl.ds(b * 3 * EMB + 2 * EMB + h, 16)] = (
                    (xa - nz * r0[hi]) * iv)
            return row_carry

        lax.fori_loop(0, CHUNK, row_body, None, unroll=2)

    def fire_out(c, p):
        rb = base + c * CHUNK
        return [pltpu.async_copy(
            out_v[p], out.at[pl.ds(rb * 3 * EMB, CHUNK * 3 * EMB)],
            sem_o[p])]

    def wait_stage(p):
        pltpu.make_async_copy(title_idx.at[pl.ds(0, CHUNK)],
                              tidx_v[p], sem_s[p]).wait()
        for j in range(NGG):
            pltpu.make_async_copy(genres_bf.at[pl.ds(0, IDX_W)],
                                  gidx_v[p].at[j], sem_s[p]).wait()
        for j in range(NGATHER):
            pltpu.make_async_copy(text_idxf.at[pl.ds(0, IDX_W)],
                                  xidx_v[p].at[j], sem_s[p]).wait()

    def wait_text(p):
        for j in range(NGATHER):
            pltpu.make_async_copy(
                stab_v.at[xidx_v[p].at[j]],
                xrows_v[p].at[pl.ds(j * IDX_W, IDX_W)], sem_g[p]).wait()

    def wait_tg(p):
        pltpu.make_async_copy(title_tab.at[tidx_v[p]], trows_v, sem_t).wait()
        for j in range(NGG):
            pltpu.make_async_copy(
                genre_tab.at[gidx_v[p].at[j]],
                grows_v.at[pl.ds(j * IDX_W, IDX_W)], sem_t).wait()

    def wait_out(p):
        pltpu.make_async_copy(
            out_v[p], out.at[pl.ds(0, CHUNK * 3 * EMB)], sem_o[p]).wait()

    fire_stage(0, 0)
    wait_stage(0)
    fire_text_gathers(0)
    fire_tg_gathers(0)
    fire_stage(1, 1)

    def pair_body(k, pair_carry):
        for sub in range(2):
            p = sub
            q = 1 - sub
            c = k * 2 + sub

            @pl.when(c + 1 < NCHUNKS)
            def _pf():
                wait_stage(q)
                fire_text_gathers(q)

            wait_text(p)
            wait_tg(p)

            @pl.when(c >= 2)
            def _wo():
                wait_out(p)

            compute(p)
            fire_out(c, p)

            @pl.when(c + 2 < NCHUNKS)
            def _st():
                fire_stage(c + 2, p)

            @pl.when(c + 1 < NCHUNKS)
            def _tg():
                fire_tg_gathers(q)

        return pair_carry

    lax.fori_loop(0, NCHUNKS // 2, pair_body, None)
    wait_out(0)
    wait_out(1)


@jax.jit
def _run(title_idx, genres_bf, text_idxf, title_tab, genre_tab, text_tab):
    mesh = plsc.VectorSubcoreMesh(core_axis_name="c", subcore_axis_name="s")
    fn = pl.kernel(
        _body,
        out_type=jax.ShapeDtypeStruct((B * 3 * EMB,), jnp.float32),
        mesh=mesh,
        scratch_types=[
            [pltpu.VMEM((CHUNK,), jnp.int32)] * 2,
            [pltpu.VMEM((NGG, IDX_W), jnp.int32)] * 2,
            [pltpu.VMEM((NGATHER, IDX_W), jnp.int32)] * 2,
            pltpu.VMEM((CHUNK, EMB), jnp.float32),
            pltpu.VMEM((CHUNK * N_GENRES, EMB), jnp.float32),
            [pltpu.VMEM((TOK_PER_CHUNK, EMB), jnp.float32)] * 2,
            pltpu.VMEM((CHUNK * IVW,), jnp.float32),
            pltpu.VMEM((CHUNK * IVW,), jnp.float32),
            pltpu.VMEM((8, EMB), jnp.float32),
            pltpu.VMEM_SHARED((10000, EMB), jnp.float32),
            [pltpu.VMEM((CHUNK * 3 * EMB,), jnp.float32)] * 2,
            [pltpu.SemaphoreType.DMA] * 2,
            [pltpu.SemaphoreType.DMA] * 2,
            pltpu.SemaphoreType.DMA,
            [pltpu.SemaphoreType.DMA] * 2,
        ],
        compiler_params=pltpu.CompilerParams(needs_layout_passes=False,
                                             use_tc_tiling_on_sc=False),
    )
    return fn(title_idx, genres_bf, text_idxf, title_tab, genre_tab, text_tab)


def kernel(movie_title, movie_genres, movie_title_text,
           title_table, genre_table, text_table):
    title_idx = movie_title.astype(jnp.int32)
    genres_bf = movie_genres.astype(jnp.int32).reshape(-1)
    text_idxf = movie_title_text.astype(jnp.int32).reshape(-1)
    flat = _run(title_idx, genres_bf, text_idxf,
                title_table, genre_table, text_table)
    return flat.reshape(B, 3 * EMB)
